# Initial kernel scaffold; baseline (speedup 1.0000x reference)
import jax
import jax.numpy as jnp
from jax.experimental import pallas as pl


def kernel(xyz, features, W1, b1, W2, b2, W3, b3):
    raise NotImplementedError("write your pallas kernel here")


# speedup vs baseline: 5.3598x; 5.3598x over previous
import functools

import numpy as np
import jax
import jax.numpy as jnp
from jax.experimental import pallas as pl

_pallas_call = pl.pallas_call

NPOINT = 1024
RADIUS = 0.2
NSAMPLE = 32


def _fps_body(xyz_ref, newx_ref):
    x = xyz_ref[0]
    y = xyz_ref[1]
    z = xyz_ref[2]
    B, N = x.shape
    S = newx_ref.shape[2]
    lane_n = jax.lax.broadcasted_iota(jnp.int32, (1, N), 1)
    lane_s = jax.lax.broadcasted_iota(jnp.int32, (1, S), 1)

    def body(i, carry):
        dists, cx, cy, cz = carry
        sel_s = lane_s == i
        newx_ref[0] = jnp.where(sel_s, cx, newx_ref[0])
        newx_ref[1] = jnp.where(sel_s, cy, newx_ref[1])
        newx_ref[2] = jnp.where(sel_s, cz, newx_ref[2])
        dx = x - cx
        dy = y - cy
        dz = z - cz
        d = dx * dx + dy * dy + dz * dz
        dists = jnp.minimum(dists, d)
        m = jnp.max(dists, axis=1, keepdims=True)
        cand = jnp.where(dists == m, lane_n, N)
        far = jnp.min(cand, axis=1, keepdims=True)
        sel = lane_n == far
        cx = jnp.sum(jnp.where(sel, x, 0.0), axis=1, keepdims=True)
        cy = jnp.sum(jnp.where(sel, y, 0.0), axis=1, keepdims=True)
        cz = jnp.sum(jnp.where(sel, z, 0.0), axis=1, keepdims=True)
        return dists, cx, cy, cz

    dists0 = jnp.full((B, N), 1e10, dtype=jnp.float32)
    jax.lax.fori_loop(0, S, body, (dists0, x[:, :1], y[:, :1], z[:, :1]))


def _bq_body(xyz_ref, nx_ref, tri_ref, idx_ref, *, n, nsample, radius_sq, chunk):
    b = pl.program_id(0)
    tri = tri_ref[...]
    rb = nx_ref.shape[1]
    nx = nx_ref[0]
    cx = nx[:, 0:1]
    cy = nx[:, 1:2]
    cz = nx[:, 2:3]
    nch = n // chunk
    lane_c = jax.lax.broadcasted_iota(jnp.int32, (1, chunk), 1)
    svals = jax.lax.broadcasted_iota(jnp.int32, (1, nsample, 1), 1) + 1

    def cond(carry):
        c, cnt, _ = carry
        return jnp.logical_and(c < nch, jnp.min(cnt) < nsample)

    def body(carry):
        c, cnt, acc = carry
        jb = pl.multiple_of(c * chunk, chunk)
        xs = xyz_ref[0, 0:1, pl.ds(jb, chunk)]
        ys = xyz_ref[0, 1:2, pl.ds(jb, chunk)]
        zs = xyz_ref[0, 2:3, pl.ds(jb, chunk)]
        dx = xs - cx
        dy = ys - cy
        dz = zs - cz
        d = dx * dx + dy * dy + dz * dz
        m = d <= radius_sq
        pos = jnp.dot(m.astype(jnp.float32), tri,
                      preferred_element_type=jnp.float32).astype(jnp.int32)
        gpos = pos + cnt
        sval = jnp.where(jnp.logical_and(m, gpos <= nsample), gpos, 0)
        jglob = lane_c + jb
        e = sval[:, None, :] == svals
        contrib = jnp.sum(jnp.where(e, jglob[:, None, :], 0), axis=2)
        return c + 1, cnt + pos[:, chunk - 1:chunk], acc + contrib

    cnt0 = jnp.zeros((rb, 1), jnp.int32)
    acc0 = jnp.zeros((rb, nsample), jnp.int32)
    _, cnt, acc = jax.lax.while_loop(cond, body, (jnp.int32(0), cnt0, acc0))
    srange = jax.lax.broadcasted_iota(jnp.int32, (rb, nsample), 1)
    out = jnp.where(srange < cnt, acc, acc[:, 0:1])
    idx_ref[0] = out + b * n


def _g_body(xyz_ref, feat_ref, w1a_ref, w1b_ref, b1_ref, g_ref):
    g_ref[0] = (
        jnp.dot(xyz_ref[0], w1a_ref[...], preferred_element_type=jnp.float32)
        + jnp.dot(feat_ref[0], w1b_ref[...], preferred_element_type=jnp.float32)
        + b1_ref[...]
    )


def _mlp_body(gg_ref, nx_ref, w1a_ref, w2_ref, b2_ref, w3_ref, b3_ref,
              out_ref, *, nsample):
    rbm = nx_ref.shape[1]
    q = jnp.dot(nx_ref[0], w1a_ref[...], preferred_element_type=jnp.float32)
    g = gg_ref[...]
    c = g.shape[1]
    h1 = jnp.maximum(g.reshape(rbm, nsample, c) - q[:, None, :], 0.0)
    h1 = h1.reshape(rbm * nsample, c)
    h2 = jnp.maximum(
        jnp.dot(h1, w2_ref[...], preferred_element_type=jnp.float32)
        + b2_ref[...], 0.0)
    h3 = jnp.maximum(
        jnp.dot(h2, w3_ref[...], preferred_element_type=jnp.float32)
        + b3_ref[...], 0.0)
    f = h3.shape[1]
    out_ref[0] = jnp.max(h3.reshape(rbm, nsample, f), axis=1)


def _run(xyz, features, W1, b1, W2, b2, W3, b3, *,
         npoint, radius, nsample, rb, chunk, rbm):
    B, N, _ = xyz.shape
    C = features.shape[2]
    F3 = W3.shape[1]

    xyzT = jnp.transpose(xyz, (2, 0, 1))
    newxT = _pallas_call(
        _fps_body,
        out_shape=jax.ShapeDtypeStruct((3, B, npoint), jnp.float32),
    )(xyzT)
    new_xyz = jnp.transpose(newxT, (1, 2, 0))

    xyzB = jnp.transpose(xyz, (0, 2, 1))
    r2 = np.float32(radius ** 2)
    tri = (jnp.arange(chunk, dtype=jnp.int32)[:, None]
           <= jnp.arange(chunk, dtype=jnp.int32)[None, :]).astype(jnp.float32)
    bq = functools.partial(_bq_body, n=N, nsample=nsample, radius_sq=r2,
                           chunk=chunk)
    idx = _pallas_call(
        bq,
        grid=(B, npoint // rb),
        in_specs=[
            pl.BlockSpec((1, 3, N), lambda b, s: (b, 0, 0)),
            pl.BlockSpec((1, rb, 3), lambda b, s: (b, s, 0)),
            pl.BlockSpec((chunk, chunk), lambda b, s: (0, 0)),
        ],
        out_specs=pl.BlockSpec((1, rb, nsample), lambda b, s: (b, s, 0)),
        out_shape=jax.ShapeDtypeStruct((B, npoint, nsample), jnp.int32),
    )(xyzB, new_xyz, tri)

    W1a = W1[:3]
    W1b = W1[3:]
    b1r = b1.reshape(1, -1)
    G = _pallas_call(
        _g_body,
        grid=(B,),
        in_specs=[
            pl.BlockSpec((1, N, 3), lambda b: (b, 0, 0)),
            pl.BlockSpec((1, N, C), lambda b: (b, 0, 0)),
            pl.BlockSpec(W1a.shape, lambda b: (0, 0)),
            pl.BlockSpec(W1b.shape, lambda b: (0, 0)),
            pl.BlockSpec(b1r.shape, lambda b: (0, 0)),
        ],
        out_specs=pl.BlockSpec((1, N, C), lambda b: (b, 0, 0)),
        out_shape=jax.ShapeDtypeStruct((B, N, C), jnp.float32),
    )(xyz, features, W1a, W1b, b1r)

    Gg = G.reshape(B * N, C)[idx.reshape(-1)]

    b2r = b2.reshape(1, -1)
    b3r = b3.reshape(1, -1)
    nblk = npoint // rbm
    mlp = functools.partial(_mlp_body, nsample=nsample)
    out = _pallas_call(
        mlp,
        grid=(B, nblk),
        in_specs=[
            pl.BlockSpec((rbm * nsample, C), lambda b, s, nblk=nblk: (b * nblk + s, 0)),
            pl.BlockSpec((1, rbm, 3), lambda b, s: (b, s, 0)),
            pl.BlockSpec(W1a.shape, lambda b, s: (0, 0)),
            pl.BlockSpec(W2.shape, lambda b, s: (0, 0)),
            pl.BlockSpec(b2r.shape, lambda b, s: (0, 0)),
            pl.BlockSpec(W3.shape, lambda b, s: (0, 0)),
            pl.BlockSpec(b3r.shape, lambda b, s: (0, 0)),
        ],
        out_specs=pl.BlockSpec((1, rbm, F3), lambda b, s: (b, s, 0)),
        out_shape=jax.ShapeDtypeStruct((B, npoint, F3), jnp.float32),
    )(Gg, new_xyz, W1a, W2, b2r, W3, b3r)

    return new_xyz, jnp.transpose(out, (0, 2, 1))


def kernel(xyz, features, W1, b1, W2, b2, W3, b3):
    return _run(xyz, features, W1, b1, W2, b2, W3, b3,
                npoint=NPOINT, radius=RADIUS, nsample=NSAMPLE,
                rb=8, chunk=512, rbm=256)


# speedup vs baseline: 6.3831x; 1.1909x over previous
import functools

import numpy as np
import jax
import jax.numpy as jnp
from jax import lax
from jax.experimental import pallas as pl
from jax.experimental.pallas import tpu as pltpu
from jax.experimental.pallas import tpu_sc as plsc

_pallas_call = pl.pallas_call

NPOINT = 1024
RADIUS = 0.2
NSAMPLE = 32


def _fps_body(xyz_ref, newx_ref):
    x = xyz_ref[0]
    y = xyz_ref[1]
    z = xyz_ref[2]
    B, N = x.shape
    S = newx_ref.shape[2]
    lane_n = jax.lax.broadcasted_iota(jnp.int32, (1, N), 1)
    lane_s = jax.lax.broadcasted_iota(jnp.int32, (1, S), 1)

    def body(i, carry):
        dists, cx, cy, cz = carry
        sel_s = lane_s == i
        newx_ref[0] = jnp.where(sel_s, cx, newx_ref[0])
        newx_ref[1] = jnp.where(sel_s, cy, newx_ref[1])
        newx_ref[2] = jnp.where(sel_s, cz, newx_ref[2])
        dx = x - cx
        dy = y - cy
        dz = z - cz
        d = dx * dx + dy * dy + dz * dz
        dists = jnp.minimum(dists, d)
        m = jnp.max(dists, axis=1, keepdims=True)
        cand = jnp.where(dists == m, lane_n, N)
        far = jnp.min(cand, axis=1, keepdims=True)
        sel = lane_n == far
        cx = jnp.sum(jnp.where(sel, x, 0.0), axis=1, keepdims=True)
        cy = jnp.sum(jnp.where(sel, y, 0.0), axis=1, keepdims=True)
        cz = jnp.sum(jnp.where(sel, z, 0.0), axis=1, keepdims=True)
        return dists, cx, cy, cz

    dists0 = jnp.full((B, N), 1e10, dtype=jnp.float32)
    jax.lax.fori_loop(0, S, body, (dists0, x[:, :1], y[:, :1], z[:, :1]))


def _bq_body(xyz_ref, nx_ref, tri_ref, idx_ref, *, n, nsample, radius_sq, chunk):
    b = pl.program_id(0)
    tri = tri_ref[...]
    rb = nx_ref.shape[1]
    nx = nx_ref[0]
    cx = nx[:, 0:1]
    cy = nx[:, 1:2]
    cz = nx[:, 2:3]
    nch = n // chunk
    lane_c = jax.lax.broadcasted_iota(jnp.int32, (1, chunk), 1)
    svals = jax.lax.broadcasted_iota(jnp.int32, (1, nsample, 1), 1) + 1

    def cond(carry):
        c, cnt, _ = carry
        return jnp.logical_and(c < nch, jnp.min(cnt) < nsample)

    def body(carry):
        c, cnt, acc = carry
        jb = pl.multiple_of(c * chunk, chunk)
        xs = xyz_ref[0, 0:1, pl.ds(jb, chunk)]
        ys = xyz_ref[0, 1:2, pl.ds(jb, chunk)]
        zs = xyz_ref[0, 2:3, pl.ds(jb, chunk)]
        dx = xs - cx
        dy = ys - cy
        dz = zs - cz
        d = dx * dx + dy * dy + dz * dz
        m = d <= radius_sq
        pos = jnp.dot(m.astype(jnp.float32), tri,
                      preferred_element_type=jnp.float32).astype(jnp.int32)
        gpos = pos + cnt
        sval = jnp.where(jnp.logical_and(m, gpos <= nsample), gpos, 0)
        jglob = lane_c + jb
        e = sval[:, None, :] == svals
        contrib = jnp.sum(jnp.where(e, jglob[:, None, :], 0), axis=2)
        return c + 1, cnt + pos[:, chunk - 1:chunk], acc + contrib

    cnt0 = jnp.zeros((rb, 1), jnp.int32)
    acc0 = jnp.zeros((rb, nsample), jnp.int32)
    _, cnt, acc = jax.lax.while_loop(cond, body, (jnp.int32(0), cnt0, acc0))
    srange = jax.lax.broadcasted_iota(jnp.int32, (rb, nsample), 1)
    out = jnp.where(srange < cnt, acc, acc[:, 0:1])
    idx_ref[0] = out + b * n


def _sc_gather(table, idxf, *, chunk=128):
    _, C = table.shape
    M = idxf.shape[0]
    info = plsc.get_sparse_core_info()
    nw = info.num_cores * info.num_subcores
    per_w = M // nw
    nch = per_w // chunk
    mesh = plsc.VectorSubcoreMesh(core_axis_name="c", subcore_axis_name="s")

    @functools.partial(
        pl.kernel, mesh=mesh,
        out_type=jax.ShapeDtypeStruct((M, C), jnp.float32),
        scratch_types=[
            pltpu.VMEM((2, chunk), jnp.int32),
            pltpu.VMEM((2, chunk, C), jnp.float32),
            pltpu.SemaphoreType.DMA,
        ],
    )
    def k(table_hbm, idx_hbm, out_hbm, idx_v, rows_v, gsem):
        wid = lax.axis_index("s") * info.num_cores + lax.axis_index("c")
        base = wid * per_w

        def issue(i, slot):
            off = base + i * chunk
            pltpu.sync_copy(idx_hbm.at[pl.ds(off, chunk)], idx_v.at[slot])
            pltpu.async_copy(table_hbm.at[idx_v.at[slot]], rows_v.at[slot],
                             gsem)

        issue(0, 0)

        def body(i, _):
            slot = lax.rem(i, 2)

            @pl.when(i + 1 < nch)
            def _():
                issue(i + 1, 1 - slot)

            pltpu.make_async_copy(table_hbm.at[idx_v.at[slot]],
                                  rows_v.at[slot], gsem).wait()
            off = base + i * chunk
            pltpu.sync_copy(rows_v.at[slot], out_hbm.at[pl.ds(off, chunk)])
            return 0

        lax.fori_loop(0, nch, body, 0)

    return k(table, idxf)


def _g_body(xyz_ref, feat_ref, w1a_ref, w1b_ref, b1_ref, g_ref):
    g = (
        jnp.dot(xyz_ref[0], w1a_ref[...], preferred_element_type=jnp.float32)
        + jnp.dot(feat_ref[0], w1b_ref[...], preferred_element_type=jnp.float32)
        + b1_ref[...]
    )
    g_ref[0] = jnp.concatenate([g, jnp.zeros_like(g)], axis=1)


def _mlp_body(gg_ref, nx_ref, w1a_ref, w2_ref, b2_ref, w3_ref, b3_ref,
              out_ref, *, nsample):
    rbm = nx_ref.shape[1]
    q = jnp.dot(nx_ref[0], w1a_ref[...], preferred_element_type=jnp.float32)
    c = w1a_ref.shape[1]
    g = gg_ref[:, :c]
    h1 = jnp.maximum(g.reshape(rbm, nsample, c) - q[:, None, :], 0.0)
    h1 = h1.reshape(rbm * nsample, c)
    h2 = jnp.maximum(
        jnp.dot(h1, w2_ref[...], preferred_element_type=jnp.float32)
        + b2_ref[...], 0.0)
    h3 = jnp.maximum(
        jnp.dot(h2, w3_ref[...], preferred_element_type=jnp.float32)
        + b3_ref[...], 0.0)
    f = h3.shape[1]
    out_ref[0] = jnp.max(h3.reshape(rbm, nsample, f), axis=1)


def _run(xyz, features, W1, b1, W2, b2, W3, b3, *,
         npoint, radius, nsample, rb, chunk, rbm):
    B, N, _ = xyz.shape
    C = features.shape[2]
    F3 = W3.shape[1]

    xyzT = jnp.transpose(xyz, (2, 0, 1))
    newxT = _pallas_call(
        _fps_body,
        out_shape=jax.ShapeDtypeStruct((3, B, npoint), jnp.float32),
    )(xyzT)
    new_xyz = jnp.transpose(newxT, (1, 2, 0))

    xyzB = jnp.transpose(xyz, (0, 2, 1))
    r2 = np.float32(radius ** 2)
    tri = (jnp.arange(chunk, dtype=jnp.int32)[:, None]
           <= jnp.arange(chunk, dtype=jnp.int32)[None, :]).astype(jnp.float32)
    bq = functools.partial(_bq_body, n=N, nsample=nsample, radius_sq=r2,
                           chunk=chunk)
    idx = _pallas_call(
        bq,
        grid=(B, npoint // rb),
        in_specs=[
            pl.BlockSpec((1, 3, N), lambda b, s: (b, 0, 0)),
            pl.BlockSpec((1, rb, 3), lambda b, s: (b, s, 0)),
            pl.BlockSpec((chunk, chunk), lambda b, s: (0, 0)),
        ],
        out_specs=pl.BlockSpec((1, rb, nsample), lambda b, s: (b, s, 0)),
        out_shape=jax.ShapeDtypeStruct((B, npoint, nsample), jnp.int32),
    )(xyzB, new_xyz, tri)

    W1a = W1[:3]
    W1b = W1[3:]
    b1r = b1.reshape(1, -1)
    G = _pallas_call(
        _g_body,
        grid=(B,),
        in_specs=[
            pl.BlockSpec((1, N, 3), lambda b: (b, 0, 0)),
            pl.BlockSpec((1, N, C), lambda b: (b, 0, 0)),
            pl.BlockSpec(W1a.shape, lambda b: (0, 0)),
            pl.BlockSpec(W1b.shape, lambda b: (0, 0)),
            pl.BlockSpec(b1r.shape, lambda b: (0, 0)),
        ],
        out_specs=pl.BlockSpec((1, N, 2 * C), lambda b: (b, 0, 0)),
        out_shape=jax.ShapeDtypeStruct((B, N, 2 * C), jnp.float32),
    )(xyz, features, W1a, W1b, b1r)

    Gg = _sc_gather(G.reshape(B * N, 2 * C), idx.reshape(-1))

    b2r = b2.reshape(1, -1)
    b3r = b3.reshape(1, -1)
    nblk = npoint // rbm
    mlp = functools.partial(_mlp_body, nsample=nsample)
    out = _pallas_call(
        mlp,
        grid=(B, nblk),
        in_specs=[
            pl.BlockSpec((rbm * nsample, 2 * C), lambda b, s, nblk=nblk: (b * nblk + s, 0)),
            pl.BlockSpec((1, rbm, 3), lambda b, s: (b, s, 0)),
            pl.BlockSpec(W1a.shape, lambda b, s: (0, 0)),
            pl.BlockSpec(W2.shape, lambda b, s: (0, 0)),
            pl.BlockSpec(b2r.shape, lambda b, s: (0, 0)),
            pl.BlockSpec(W3.shape, lambda b, s: (0, 0)),
            pl.BlockSpec(b3r.shape, lambda b, s: (0, 0)),
        ],
        out_specs=pl.BlockSpec((1, rbm, F3), lambda b, s: (b, s, 0)),
        out_shape=jax.ShapeDtypeStruct((B, npoint, F3), jnp.float32),
    )(Gg, new_xyz, W1a, W2, b2r, W3, b3r)

    return new_xyz, jnp.transpose(out, (0, 2, 1))


def kernel(xyz, features, W1, b1, W2, b2, W3, b3):
    return _run(xyz, features, W1, b1, W2, b2, W3, b3,
                npoint=NPOINT, radius=RADIUS, nsample=NSAMPLE,
                rb=8, chunk=512, rbm=256)
